# Initial kernel scaffold; baseline (speedup 1.0000x reference)
import jax
import jax.numpy as jnp
from jax.experimental import pallas as pl


def kernel(x, duration, max_len):
    raise NotImplementedError("write your pallas kernel here")


# speedup vs baseline: 62.2774x; 62.2774x over previous
import functools

import jax
import jax.numpy as jnp
from jax import lax
from jax.experimental import pallas as pl
from jax.experimental.pallas import tpu as pltpu
from jax.experimental.pallas import tpu_sc as plsc

B = 16
P = 512
D = 256
T = 2048
L = 16
HALF = T // 2
C = 128
NCH = HALF // C


def _zero_rows(gbuf, r_lo, r_hi):
    zeros = jnp.zeros((L,), jnp.float32)

    def body(r, _):
        for k in range(D // L):
            gbuf[r, pl.ds(k * L, L)] = zeros
        return 0

    lax.fori_loop(r_lo, r_hi, body, 0)


def _lr_body(x_hbm, dur_hbm, out_hbm, mel_hbm,
             dur_v, map_v, idx_v, gbuf, mel_v, sem):
    b = lax.axis_index("s")
    h = lax.axis_index("c")

    pltpu.sync_copy(dur_hbm.at[b], dur_v)

    zi = jnp.zeros((L,), jnp.int32)

    def zmap(i, _):
        map_v[pl.ds(i * L, L)] = zi
        return 0

    lax.fori_loop(0, T // L, zmap, 0)

    carry = jnp.int32(0)
    for i in range(P // L):
        d = dur_v[pl.ds(i * L, L)]
        s = plsc.cumsum(d) + carry
        start = s - d
        pvec = lax.iota(jnp.int32, 16) + (i * L)
        msk = (d > 0) & (start < T)
        plsc.store_scatter(map_v, [start], pvec, mask=msk)
        carry = jnp.max(s)
    mel_len = carry

    base = b * P

    def cmx(i, mc):
        v = map_v[pl.ds(i * L, L)]
        s = jnp.maximum(plsc.cummax(v), mc)
        idx_v[pl.ds(i * L, L)] = jnp.minimum(s, P - 1) + base
        return jnp.max(s)

    lax.fori_loop(0, T // L, cmx, jnp.int32(0))

    nv = jnp.clip(mel_len - h * HALF, 0, HALF)
    row0 = b * T + h * HALF
    for j in range(NCH):
        lo = j * C
        nvj = nv - lo

        @pl.when(nvj > 0)
        def _gather():
            idx_slice = idx_v.at[pl.ds(h * HALF + lo, C)]
            pltpu.async_copy(x_hbm.at[idx_slice], gbuf, sem).wait()

        @pl.when((nvj > 0) & (nvj < C))
        def _zero_tail():
            _zero_rows(gbuf, nvj, C)

        @pl.when(nvj == 0)
        def _zero_all():
            _zero_rows(gbuf, 0, C)

        pltpu.sync_copy(gbuf, out_hbm.at[pl.ds(row0 + lo, C)])

        @pl.when((nvj > 0) & (nvj < C))
        def _zero_head():
            _zero_rows(gbuf, 0, nvj)

    @pl.when(h == 0)
    def _write_mel():
        mel_v[...] = jnp.full((L,), mel_len, jnp.int32)
        pltpu.sync_copy(mel_v, mel_hbm.at[b])


@jax.jit
def _length_regulate(x_flat, duration):
    mesh = plsc.VectorSubcoreMesh(core_axis_name="c", subcore_axis_name="s")
    out, mel = pl.kernel(
        _lr_body,
        out_type=[
            jax.ShapeDtypeStruct((B * T, D), jnp.float32),
            jax.ShapeDtypeStruct((B, L), jnp.int32),
        ],
        mesh=mesh,
        compiler_params=pltpu.CompilerParams(needs_layout_passes=False),
        scratch_types=[
            pltpu.VMEM((P,), jnp.int32),
            pltpu.VMEM((T,), jnp.int32),
            pltpu.VMEM((T,), jnp.int32),
            pltpu.VMEM((C, D), jnp.float32),
            pltpu.VMEM((L,), jnp.int32),
            pltpu.SemaphoreType.DMA,
        ],
    )(x_flat, duration)
    return out, mel


def kernel(x, duration, max_len):
    x_flat = x.reshape(B * P, D)
    out, mel = _length_regulate(x_flat, duration.astype(jnp.int32))
    return out.reshape(B, T, D), mel[:, 0]


# speedup vs baseline: 68.4377x; 1.0989x over previous
import functools

import jax
import jax.numpy as jnp
from jax import lax
from jax.experimental import pallas as pl
from jax.experimental.pallas import tpu as pltpu
from jax.experimental.pallas import tpu_sc as plsc

B = 16
P = 512
D = 256
T = 2048
L = 16
HALF = T // 2
C = 128
NCH = HALF // C


def _zero_rows(buf, r_lo, r_hi):
    zeros = jnp.zeros((L,), jnp.float32)

    def body(r, _):
        for k in range(D // L):
            buf[r, pl.ds(k * L, L)] = zeros
        return 0

    lax.fori_loop(r_lo, r_hi, body, 0)


def _lr_body(x_hbm, dur_hbm, out_hbm, mel_hbm,
             dur_v, map_v, idx_v, gbuf, mel_v, gsem, wsem0, wsem1):
    b = lax.axis_index("s")
    h = lax.axis_index("c")

    pltpu.sync_copy(dur_hbm.at[b], dur_v)

    zi = jnp.zeros((L,), jnp.int32)

    def zmap(i, _):
        map_v[pl.ds(i * L, L)] = zi
        return 0

    lax.fori_loop(0, T // L, zmap, 0)

    carry = jnp.int32(0)
    for i in range(P // L):
        d = dur_v[pl.ds(i * L, L)]
        s = plsc.cumsum(d) + carry
        start = s - d
        pvec = lax.iota(jnp.int32, 16) + (i * L)
        msk = (d > 0) & (start < T)
        plsc.store_scatter(map_v, [start], pvec, mask=msk)
        carry = jnp.max(s)
    mel_len = carry

    base = b * P

    def cmx(i, mc):
        v = map_v[pl.ds(i * L, L)]
        s = jnp.maximum(plsc.cummax(v), mc)
        idx_v[pl.ds(i * L, L)] = jnp.minimum(s, P - 1) + base
        return jnp.max(s)

    lax.fori_loop(0, T // L, cmx, jnp.int32(0))

    nv = jnp.clip(mel_len - h * HALF, 0, HALF)
    row0 = b * T + h * HALF
    bufs = (gbuf.at[0], gbuf.at[1])
    wsems = (wsem0, wsem1)
    for j in range(NCH):
        q = j & 1
        nvj = nv - j * C
        zc = jnp.clip(nvj, 0, C)
        prev = jnp.clip(nvj + 2 * C, 0, C)

        if j >= 2:
            pltpu.make_async_copy(
                bufs[q], out_hbm.at[pl.ds(row0 + (j - 2) * C, C)], wsems[q]
            ).wait()

        @pl.when(nvj > 0)
        def _gather():
            idx_slice = idx_v.at[pl.ds(h * HALF + j * C, C)]
            pltpu.async_copy(x_hbm.at[idx_slice], bufs[q], gsem).wait()

        _zero_rows(bufs[q], zc, prev)

        pltpu.async_copy(bufs[q], out_hbm.at[pl.ds(row0 + j * C, C)], wsems[q])

    for j in (NCH - 2, NCH - 1):
        q = j & 1
        pltpu.make_async_copy(
            bufs[q], out_hbm.at[pl.ds(row0 + j * C, C)], wsems[q]
        ).wait()

    @pl.when(h == 0)
    def _write_mel():
        mel_v[...] = jnp.full((L,), mel_len, jnp.int32)
        pltpu.sync_copy(mel_v, mel_hbm.at[b])


@jax.jit
def _length_regulate(x_flat, duration):
    mesh = plsc.VectorSubcoreMesh(core_axis_name="c", subcore_axis_name="s")
    out, mel = pl.kernel(
        _lr_body,
        out_type=[
            jax.ShapeDtypeStruct((B * T, D), jnp.float32),
            jax.ShapeDtypeStruct((B, L), jnp.int32),
        ],
        mesh=mesh,
        compiler_params=pltpu.CompilerParams(needs_layout_passes=False),
        scratch_types=[
            pltpu.VMEM((P,), jnp.int32),
            pltpu.VMEM((T,), jnp.int32),
            pltpu.VMEM((T,), jnp.int32),
            pltpu.VMEM((2, C, D), jnp.float32),
            pltpu.VMEM((L,), jnp.int32),
            pltpu.SemaphoreType.DMA,
            pltpu.SemaphoreType.DMA,
            pltpu.SemaphoreType.DMA,
        ],
    )(x_flat, duration)
    return out, mel


def kernel(x, duration, max_len):
    x_flat = x.reshape(B * P, D)
    out, mel = _length_regulate(x_flat, duration.astype(jnp.int32))
    return out.reshape(B, T, D), mel[:, 0]
